# baseline (speedup 1.0000x reference)
import functools
import math

import jax
import jax.numpy as jnp
from jax import lax
from jax.experimental import pallas as pl
from jax.experimental.pallas import tpu as pltpu
from jax.experimental.pallas import tpu_sc as plsc

MUL = 128
HID = 64
NATTR = 10
NPASS = 5
GW = 2 * HID
AVG_NUM_NEIGHBORS = 32.0
_L_OF_D = (0, 1, 1, 1, 2, 2, 2, 2, 2)


def _h_body(ed_ref, w0_ref, w1_ref, w2_ref, h_ref):
    h = jax.nn.silu(jnp.dot(ed_ref[...], w0_ref[...],
                            preferred_element_type=jnp.float32) * (1.0 / math.sqrt(8.0)))
    h = jax.nn.silu(jnp.dot(h, w1_ref[...],
                            preferred_element_type=jnp.float32) * (1.0 / math.sqrt(HID)))
    h_ref[...] = jax.nn.silu(jnp.dot(h, w2_ref[...],
                                     preferred_element_type=jnp.float32) * (1.0 / math.sqrt(HID)))


def _expand_body_pair(h_ref, sh2_ref, g_ref):
    h = h_ref[...]
    sh = sh2_ref[...]
    g_ref[...] = jnp.concatenate([sh[:, 0:1] * h, sh[:, 1:2] * h], axis=1)


def _expand_body_half(h_ref, sh1_ref, g_ref):
    h = h_ref[...]
    g_ref[...] = jnp.concatenate([sh1_ref[...] * h, jnp.zeros_like(h)], axis=1)


def _build_sc_scatter(E, N):
    n_tiles = 32
    ept = E // n_tiles
    C = 40
    n_chunks = ept // C
    NBUF = 5
    n_outer = n_chunks // NBUF
    rows = (N + 127) // 128 * 128 + 128
    rows_per_tile = rows // 16
    mesh = plsc.VectorSubcoreMesh(core_axis_name="c", subcore_axis_name="s")

    @functools.partial(
        pl.kernel,
        out_type=jax.ShapeDtypeStruct((2, rows, GW), jnp.float32),
        mesh=mesh,
        scratch_types=[pltpu.VMEM((C, GW), jnp.float32)] * NBUF
                      + [pltpu.VMEM((C,), jnp.int32)] * NBUF + [
            pltpu.VMEM_SHARED((rows, GW), jnp.float32),
        ] + [pltpu.SemaphoreType.DMA] * NBUF)
    def sc_scatter(src_hbm, g_hbm, zero_hbm, o_hbm, *rest):
        bufs = rest[:NBUF]
        idxs = rest[NBUF:2 * NBUF]
        accum = rest[2 * NBUF]
        sems = rest[2 * NBUF + 1:]
        c = lax.axis_index("c")
        s = lax.axis_index("s")
        wid = c * 16 + s
        tile_base = wid * ept
        row0 = s * rows_per_tile

        pltpu.sync_copy(zero_hbm.at[pl.ds(row0, rows_per_tile)],
                        accum.at[pl.ds(row0, rows_per_tile)])
        plsc.subcore_barrier()

        def fetch(i, b):
            pltpu.async_copy(src_hbm.at[wid, i], idxs[b], sems[b])
            pltpu.async_copy(g_hbm.at[pl.ds(tile_base + i * C, C)],
                             bufs[b], sems[b])

        for b in range(NBUF):
            fetch(b, b)

        def outer(j, carry):
            for b in range(NBUF):
                i = j * NBUF + b
                pltpu.make_async_copy(src_hbm.at[0, 0], idxs[b], sems[b]).wait()
                pltpu.make_async_copy(g_hbm.at[pl.ds(0, C)], bufs[b],
                                      sems[b]).wait()
                pltpu.sync_copy(bufs[b], accum.at[idxs[b]], add=True)

                @pl.when(j < n_outer - 1)
                def _():
                    fetch(i + NBUF, b)
            return carry

        lax.fori_loop(0, n_outer, outer, 0)
        plsc.subcore_barrier()
        pltpu.sync_copy(accum.at[pl.ds(row0, rows_per_tile)],
                        o_hbm.at[c, pl.ds(row0, rows_per_tile)])

    return sc_scatter


def _sc_body(nf_ref, na_ref, wskip_ref, sc_ref):
    nf = nf_ref[...]
    acc = jnp.zeros_like(nf)
    for v in range(NATTR):
        acc = acc + jnp.dot(nf, wskip_ref[:, v, :],
                            preferred_element_type=jnp.float32) * na_ref[:, v:v + 1]
    sc_ref[...] = acc * (1.0 / math.sqrt(MUL * NATTR))


def _make_out_body(p, nd):
    def _out_body(nf_ref, ta_ref, tb_ref, wlin_ref, wmlp3_ref,
                  w20_ref, w21_ref, w22_ref, out_ref):
        x = jnp.dot(nf_ref[...], wlin_ref[...],
                    preferred_element_type=jnp.float32) * (1.0 / math.sqrt(MUL))
        w2s = (w20_ref, w21_ref, w22_ref)
        scale = 1.0 / (math.sqrt(HID) * math.sqrt(MUL) * AVG_NUM_NEIGHBORS)
        for dl in range(nd):
            d = 2 * p + dl
            l = _L_OF_D[d]
            td = (ta_ref[0, :, HID * dl:HID * (dl + 1)]
                  + tb_ref[0, :, HID * dl:HID * (dl + 1)])
            m = jnp.dot(td, wmlp3_ref[:, l * MUL:(l + 1) * MUL],
                        preferred_element_type=jnp.float32)
            out_ref[dl] = jnp.dot(x * m, w2s[l][...],
                                  preferred_element_type=jnp.float32) * scale
    return _out_body


def kernel(node_feat, node_attr, edge_idx, edge_dist_embedding, edge_diff_embedding,
           W_skip, W_lin1, W_mlp0, W_mlp1, W_mlp2, W_mlp3, W2_0, W2_1, W2_2):
    N = node_feat.shape[0]
    E = edge_dist_embedding.shape[0]
    src = edge_idx[:, 0]
    rows = (N + 127) // 128 * 128 + 128

    EB = 2560
    h = pl.pallas_call(
        _h_body,
        grid=(E // EB,),
        in_specs=[
            pl.BlockSpec((EB, 8), lambda i: (i, 0)),
            pl.BlockSpec((8, HID), lambda i: (0, 0)),
            pl.BlockSpec((HID, HID), lambda i: (0, 0)),
            pl.BlockSpec((HID, HID), lambda i: (0, 0)),
        ],
        out_specs=pl.BlockSpec((EB, HID), lambda i: (i, 0)),
        out_shape=jax.ShapeDtypeStruct((E, HID), jnp.float32),
    )(edge_dist_embedding, W_mlp0, W_mlp1, W_mlp2)

    def expand(p):
        nd = 2 if p < 4 else 1
        body = _expand_body_pair if nd == 2 else _expand_body_half
        return pl.pallas_call(
            body,
            grid=(E // EB,),
            in_specs=[
                pl.BlockSpec((EB, HID), lambda i: (i, 0)),
                pl.BlockSpec((EB, nd), lambda i: (i, 0)),
            ],
            out_specs=pl.BlockSpec((EB, GW), lambda i: (i, 0)),
            out_shape=jax.ShapeDtypeStruct((E, GW), jnp.float32),
        )(h, edge_diff_embedding[:, 2 * p:2 * p + nd])

    zeros = jnp.zeros((rows, GW), jnp.float32)
    src3d = src.reshape(32, -1, 40)
    scatter = _build_sc_scatter(E, N)
    ts = [scatter(src3d, expand(p), zeros) for p in range(NPASS)]

    NB = 400
    sc = pl.pallas_call(
        _sc_body,
        grid=(N // NB,),
        in_specs=[
            pl.BlockSpec((NB, MUL), lambda i: (i, 0)),
            pl.BlockSpec((NB, NATTR), lambda i: (i, 0)),
            pl.BlockSpec((MUL, NATTR, MUL), lambda i: (0, 0, 0)),
        ],
        out_specs=pl.BlockSpec((NB, MUL), lambda i: (i, 0)),
        out_shape=jax.ShapeDtypeStruct((N, MUL), jnp.float32),
    )(node_feat, node_attr, W_skip)

    outs = []
    for p in range(NPASS):
        nd = 2 if p < 4 else 1
        outs.append(pl.pallas_call(
            _make_out_body(p, nd),
            grid=(N // NB,),
            in_specs=[
                pl.BlockSpec((NB, MUL), lambda i: (i, 0)),
                pl.BlockSpec((1, NB, GW), lambda i: (0, i, 0)),
                pl.BlockSpec((1, NB, GW), lambda i: (1, i, 0)),
                pl.BlockSpec((MUL, MUL), lambda i: (0, 0)),
                pl.BlockSpec((HID, 3 * MUL), lambda i: (0, 0)),
                pl.BlockSpec((MUL, MUL), lambda i: (0, 0)),
                pl.BlockSpec((MUL, MUL), lambda i: (0, 0)),
                pl.BlockSpec((MUL, MUL), lambda i: (0, 0)),
            ],
            out_specs=pl.BlockSpec((nd, NB, MUL), lambda i: (0, i, 0)),
            out_shape=jax.ShapeDtypeStruct((nd, N, MUL), jnp.float32),
        )(node_feat, ts[p], ts[p], W_lin1, W_mlp3, W2_0, W2_1, W2_2))

    out9 = jnp.concatenate(outs, axis=0)
    return (jnp.transpose(out9, (1, 2, 0)), sc)


# speedup vs baseline: 1.6010x; 1.6010x over previous
import functools
import math

import jax
import jax.numpy as jnp
from jax import lax
from jax.experimental import pallas as pl
from jax.experimental.pallas import tpu as pltpu
from jax.experimental.pallas import tpu_sc as plsc

MUL = 128
HID = 64
NATTR = 10
NPASS = 5
GW = 2 * HID
AVG_NUM_NEIGHBORS = 32.0
_L_OF_D = (0, 1, 1, 1, 2, 2, 2, 2, 2)


def _mlp_g_body(ed_ref, sh_ref, w0_ref, w1_ref, w2_ref, *g_refs):
    h = jax.nn.silu(jnp.dot(ed_ref[...], w0_ref[...],
                            preferred_element_type=jnp.float32) * (1.0 / math.sqrt(8.0)))
    h = jax.nn.silu(jnp.dot(h, w1_ref[...],
                            preferred_element_type=jnp.float32) * (1.0 / math.sqrt(HID)))
    h = jax.nn.silu(jnp.dot(h, w2_ref[...],
                            preferred_element_type=jnp.float32) * (1.0 / math.sqrt(HID)))
    sh = sh_ref[...]
    for p, ref in enumerate(g_refs):
        cols = []
        for d in range(2):
            dc = 2 * p + d
            cols.append(sh[:, dc:dc + 1] * h if dc < 9 else jnp.zeros_like(h))
        ref[...] = jnp.concatenate(cols, axis=1)


def _build_sc_scatter(E, N):
    n_tiles = 32
    ept = E // n_tiles
    C = 40
    n_chunks = ept // C
    NBUF = 5
    n_outer = n_chunks // NBUF
    rows = (N + 127) // 128 * 128 + 128
    rows_per_tile = rows // 16
    mesh = plsc.VectorSubcoreMesh(core_axis_name="c", subcore_axis_name="s")
    out_t = tuple(jax.ShapeDtypeStruct((2, rows, GW), jnp.float32) for _ in range(NPASS))

    @functools.partial(
        pl.kernel, out_type=out_t, mesh=mesh,
        scratch_types=[pltpu.VMEM((C, GW), jnp.float32)] * NBUF
                      + [pltpu.VMEM((C,), jnp.int32)] * NBUF + [
            pltpu.VMEM_SHARED((rows, GW), jnp.float32),
        ] + [pltpu.SemaphoreType.DMA] * NBUF)
    def sc_scatter(src_hbm, g0_hbm, g1_hbm, g2_hbm, g3_hbm, g4_hbm, zero_hbm,
                   o0, o1, o2, o3, o4, *rest):
        bufs = rest[:NBUF]
        idxs = rest[NBUF:2 * NBUF]
        accum = rest[2 * NBUF]
        sems = rest[2 * NBUF + 1:]
        c = lax.axis_index("c")
        s = lax.axis_index("s")
        wid = c * 16 + s
        tile_base = wid * ept
        row0 = s * rows_per_tile

        for g_hbm, o_hbm in ((g0_hbm, o0), (g1_hbm, o1), (g2_hbm, o2),
                             (g3_hbm, o3), (g4_hbm, o4)):
            pltpu.sync_copy(zero_hbm.at[pl.ds(row0, rows_per_tile)],
                            accum.at[pl.ds(row0, rows_per_tile)])
            plsc.subcore_barrier()

            def fetch(i, b):
                pltpu.async_copy(src_hbm.at[wid, i], idxs[b], sems[b])
                pltpu.async_copy(g_hbm.at[pl.ds(tile_base + i * C, C)],
                                 bufs[b], sems[b])

            for b in range(NBUF):
                fetch(b, b)

            def outer(j, carry):
                for b in range(NBUF):
                    i = j * NBUF + b
                    pltpu.make_async_copy(src_hbm.at[0, 0], idxs[b],
                                          sems[b]).wait()
                    pltpu.make_async_copy(g_hbm.at[pl.ds(0, C)], bufs[b],
                                          sems[b]).wait()
                    pltpu.sync_copy(bufs[b], accum.at[idxs[b]], add=True)

                    @pl.when(j < n_outer - 1)
                    def _():
                        fetch(i + NBUF, b)
                return carry

            lax.fori_loop(0, n_outer, outer, 0)
            plsc.subcore_barrier()
            pltpu.sync_copy(accum.at[pl.ds(row0, rows_per_tile)],
                            o_hbm.at[c, pl.ds(row0, rows_per_tile)])

    return sc_scatter


def _final_body(nf_ref, na_ref, *rest):
    t_refs = rest[:2 * NPASS]
    (wskip_ref, wlin_ref, wmlp3_ref, w20_ref, w21_ref, w22_ref,
     out9_ref, sc_ref) = rest[2 * NPASS:]
    nf = nf_ref[...]
    acc = jnp.zeros_like(nf)
    for v in range(NATTR):
        acc = acc + jnp.dot(nf, wskip_ref[:, v, :],
                            preferred_element_type=jnp.float32) * na_ref[:, v:v + 1]
    sc_ref[...] = acc * (1.0 / math.sqrt(MUL * NATTR))
    x = jnp.dot(nf, wlin_ref[...],
                preferred_element_type=jnp.float32) * (1.0 / math.sqrt(MUL))
    w2s = (w20_ref, w21_ref, w22_ref)
    scale = 1.0 / (math.sqrt(HID) * math.sqrt(MUL) * AVG_NUM_NEIGHBORS)
    for d in range(9):
        p, dl = divmod(d, 2)
        l = _L_OF_D[d]
        ta, tb = t_refs[2 * p], t_refs[2 * p + 1]
        td = (ta[0, :, HID * dl:HID * (dl + 1)]
              + tb[0, :, HID * dl:HID * (dl + 1)])
        m = jnp.dot(td, wmlp3_ref[:, l * MUL:(l + 1) * MUL],
                    preferred_element_type=jnp.float32)
        out9_ref[d] = jnp.dot(x * m, w2s[l][...],
                              preferred_element_type=jnp.float32) * scale


def kernel(node_feat, node_attr, edge_idx, edge_dist_embedding, edge_diff_embedding,
           W_skip, W_lin1, W_mlp0, W_mlp1, W_mlp2, W_mlp3, W2_0, W2_1, W2_2):
    N = node_feat.shape[0]
    E = edge_dist_embedding.shape[0]
    src = edge_idx[:, 0]
    rows = (N + 127) // 128 * 128 + 128

    EB = 5120
    gs = pl.pallas_call(
        _mlp_g_body,
        grid=(E // EB,),
        in_specs=[
            pl.BlockSpec((EB, 8), lambda i: (i, 0)),
            pl.BlockSpec((EB, 9), lambda i: (i, 0)),
            pl.BlockSpec((8, HID), lambda i: (0, 0)),
            pl.BlockSpec((HID, HID), lambda i: (0, 0)),
            pl.BlockSpec((HID, HID), lambda i: (0, 0)),
        ],
        out_specs=[pl.BlockSpec((EB, GW), lambda i: (i, 0))] * NPASS,
        out_shape=[jax.ShapeDtypeStruct((E, GW), jnp.float32)] * NPASS,
    )(edge_dist_embedding, edge_diff_embedding, W_mlp0, W_mlp1, W_mlp2)

    zeros = jnp.zeros((rows, GW), jnp.float32)
    ts = _build_sc_scatter(E, N)(src.reshape(32, -1, 40), *gs, zeros)

    NB = 400
    t_specs = []
    t_args = []
    for t in ts:
        t_specs += [pl.BlockSpec((1, NB, GW), lambda i: (0, i, 0)),
                    pl.BlockSpec((1, NB, GW), lambda i: (1, i, 0))]
        t_args += [t, t]
    out9, sc = pl.pallas_call(
        _final_body,
        grid=(N // NB,),
        in_specs=[
            pl.BlockSpec((NB, MUL), lambda i: (i, 0)),
            pl.BlockSpec((NB, NATTR), lambda i: (i, 0)),
        ] + t_specs + [
            pl.BlockSpec((MUL, NATTR, MUL), lambda i: (0, 0, 0)),
            pl.BlockSpec((MUL, MUL), lambda i: (0, 0)),
            pl.BlockSpec((HID, 3 * MUL), lambda i: (0, 0)),
            pl.BlockSpec((MUL, MUL), lambda i: (0, 0)),
            pl.BlockSpec((MUL, MUL), lambda i: (0, 0)),
            pl.BlockSpec((MUL, MUL), lambda i: (0, 0)),
        ],
        out_specs=[
            pl.BlockSpec((9, NB, MUL), lambda i: (0, i, 0)),
            pl.BlockSpec((NB, MUL), lambda i: (i, 0)),
        ],
        out_shape=[
            jax.ShapeDtypeStruct((9, N, MUL), jnp.float32),
            jax.ShapeDtypeStruct((N, MUL), jnp.float32),
        ],
    )(node_feat, node_attr, *t_args,
      W_skip, W_lin1, W_mlp3, W2_0, W2_1, W2_2)

    return (jnp.transpose(out9, (1, 2, 0)), sc)


# speedup vs baseline: 1.6049x; 1.0024x over previous
import functools
import math

import jax
import jax.numpy as jnp
from jax import lax
from jax.experimental import pallas as pl
from jax.experimental.pallas import tpu as pltpu
from jax.experimental.pallas import tpu_sc as plsc

MUL = 128
HID = 64
NATTR = 10
NPASS = 5
GW = 2 * HID
AVG_NUM_NEIGHBORS = 32.0
_L_OF_D = (0, 1, 1, 1, 2, 2, 2, 2, 2)


def _mlp_g_body(ed_ref, sh_ref, w0_ref, w1_ref, w2_ref, *g_refs):
    h = jax.nn.silu(jnp.dot(ed_ref[...], w0_ref[...],
                            preferred_element_type=jnp.float32) * (1.0 / math.sqrt(8.0)))
    h = jax.nn.silu(jnp.dot(h, w1_ref[...],
                            preferred_element_type=jnp.float32) * (1.0 / math.sqrt(HID)))
    h = jax.nn.silu(jnp.dot(h, w2_ref[...],
                            preferred_element_type=jnp.float32) * (1.0 / math.sqrt(HID)))
    sh = sh_ref[...]
    for p, ref in enumerate(g_refs):
        cols = []
        for d in range(2):
            dc = 2 * p + d
            cols.append(sh[:, dc:dc + 1] * h if dc < 9 else jnp.zeros_like(h))
        ref[...] = jnp.concatenate(cols, axis=1)


def _build_sc_scatter(E, N):
    n_tiles = 32
    ept = E // n_tiles
    C = 40
    n_chunks = ept // C
    NBUF = 5
    n_outer = n_chunks // NBUF
    rows = (N + 127) // 128 * 128 + 128
    rows_per_tile = rows // 16
    mesh = plsc.VectorSubcoreMesh(core_axis_name="c", subcore_axis_name="s")
    out_t = tuple(jax.ShapeDtypeStruct((2, rows, GW), jnp.float32) for _ in range(NPASS))

    @functools.partial(
        pl.kernel, out_type=out_t, mesh=mesh,
        scratch_types=[pltpu.VMEM((C, GW), jnp.float32)] * NBUF
                      + [pltpu.VMEM((C,), jnp.int32)] * NBUF + [
            pltpu.VMEM_SHARED((rows, GW), jnp.float32),
        ] + [pltpu.SemaphoreType.DMA] * NBUF)
    def sc_scatter(src_hbm, g0_hbm, g1_hbm, g2_hbm, g3_hbm, g4_hbm, zero_hbm,
                   o0, o1, o2, o3, o4, *rest):
        bufs = rest[:NBUF]
        idxs = rest[NBUF:2 * NBUF]
        accum = rest[2 * NBUF]
        sems = rest[2 * NBUF + 1:]
        c = lax.axis_index("c")
        s = lax.axis_index("s")
        wid = c * 16 + s
        tile_base = wid * ept
        row0 = s * rows_per_tile

        for g_hbm, o_hbm in ((g0_hbm, o0), (g1_hbm, o1), (g2_hbm, o2),
                             (g3_hbm, o3), (g4_hbm, o4)):
            pltpu.sync_copy(zero_hbm.at[pl.ds(row0, rows_per_tile)],
                            accum.at[pl.ds(row0, rows_per_tile)])
            plsc.subcore_barrier()

            def fetch(i, b):
                pltpu.async_copy(src_hbm.at[wid, i], idxs[b], sems[b])
                pltpu.async_copy(g_hbm.at[pl.ds(tile_base + i * C, C)],
                                 bufs[b], sems[b])

            for b in range(NBUF):
                fetch(b, b)

            def outer(j, carry):
                for b in range(NBUF):
                    i = j * NBUF + b
                    pltpu.make_async_copy(src_hbm.at[0, 0], idxs[b],
                                          sems[b]).wait()
                    pltpu.make_async_copy(g_hbm.at[pl.ds(0, C)], bufs[b],
                                          sems[b]).wait()
                    pltpu.sync_copy(bufs[b], accum.at[idxs[b]], add=True)

                    @pl.when(j < n_outer - 1)
                    def _():
                        fetch(i + NBUF, b)
                return carry

            lax.fori_loop(0, n_outer, outer, 0)
            plsc.subcore_barrier()
            pltpu.sync_copy(accum.at[pl.ds(row0, rows_per_tile)],
                            o_hbm.at[c, pl.ds(row0, rows_per_tile)])

    return sc_scatter


def _final_body(nf_ref, na_ref, *rest):
    t_refs = rest[:2 * NPASS]
    (wskip_ref, wlin_ref, wmlp3_ref, w20_ref, w21_ref, w22_ref,
     out9_ref, sc_ref) = rest[2 * NPASS:]
    nf = nf_ref[...]
    acc = jnp.zeros_like(nf)
    for v in range(NATTR):
        acc = acc + jnp.dot(nf, wskip_ref[:, v, :],
                            preferred_element_type=jnp.float32) * na_ref[:, v:v + 1]
    sc_ref[...] = acc * (1.0 / math.sqrt(MUL * NATTR))
    x = jnp.dot(nf, wlin_ref[...],
                preferred_element_type=jnp.float32) * (1.0 / math.sqrt(MUL))
    w2s = (w20_ref, w21_ref, w22_ref)
    scale = 1.0 / (math.sqrt(HID) * math.sqrt(MUL) * AVG_NUM_NEIGHBORS)
    for d in range(9):
        p, dl = divmod(d, 2)
        l = _L_OF_D[d]
        ta, tb = t_refs[2 * p], t_refs[2 * p + 1]
        td = (ta[0, :, HID * dl:HID * (dl + 1)]
              + tb[0, :, HID * dl:HID * (dl + 1)])
        m = jnp.dot(td, wmlp3_ref[:, l * MUL:(l + 1) * MUL],
                    preferred_element_type=jnp.float32)
        out9_ref[d] = jnp.dot(x * m, w2s[l][...],
                              preferred_element_type=jnp.float32) * scale


def kernel(node_feat, node_attr, edge_idx, edge_dist_embedding, edge_diff_embedding,
           W_skip, W_lin1, W_mlp0, W_mlp1, W_mlp2, W_mlp3, W2_0, W2_1, W2_2):
    N = node_feat.shape[0]
    E = edge_dist_embedding.shape[0]
    src = edge_idx[:, 0]
    rows = (N + 127) // 128 * 128 + 128

    EB = 4000
    gs = pl.pallas_call(
        _mlp_g_body,
        grid=(E // EB,),
        in_specs=[
            pl.BlockSpec((EB, 8), lambda i: (i, 0)),
            pl.BlockSpec((EB, 9), lambda i: (i, 0)),
            pl.BlockSpec((8, HID), lambda i: (0, 0)),
            pl.BlockSpec((HID, HID), lambda i: (0, 0)),
            pl.BlockSpec((HID, HID), lambda i: (0, 0)),
        ],
        out_specs=[pl.BlockSpec((EB, GW), lambda i: (i, 0))] * NPASS,
        out_shape=[jax.ShapeDtypeStruct((E, GW), jnp.float32)] * NPASS,
    )(edge_dist_embedding, edge_diff_embedding, W_mlp0, W_mlp1, W_mlp2)

    zeros = jnp.zeros((rows, GW), jnp.float32)
    ts = _build_sc_scatter(E, N)(src.reshape(32, -1, 40), *gs, zeros)

    NB = 400
    t_specs = []
    t_args = []
    for t in ts:
        t_specs += [pl.BlockSpec((1, NB, GW), lambda i: (0, i, 0)),
                    pl.BlockSpec((1, NB, GW), lambda i: (1, i, 0))]
        t_args += [t, t]
    out9, sc = pl.pallas_call(
        _final_body,
        grid=(N // NB,),
        in_specs=[
            pl.BlockSpec((NB, MUL), lambda i: (i, 0)),
            pl.BlockSpec((NB, NATTR), lambda i: (i, 0)),
        ] + t_specs + [
            pl.BlockSpec((MUL, NATTR, MUL), lambda i: (0, 0, 0)),
            pl.BlockSpec((MUL, MUL), lambda i: (0, 0)),
            pl.BlockSpec((HID, 3 * MUL), lambda i: (0, 0)),
            pl.BlockSpec((MUL, MUL), lambda i: (0, 0)),
            pl.BlockSpec((MUL, MUL), lambda i: (0, 0)),
            pl.BlockSpec((MUL, MUL), lambda i: (0, 0)),
        ],
        out_specs=[
            pl.BlockSpec((9, NB, MUL), lambda i: (0, i, 0)),
            pl.BlockSpec((NB, MUL), lambda i: (i, 0)),
        ],
        out_shape=[
            jax.ShapeDtypeStruct((9, N, MUL), jnp.float32),
            jax.ShapeDtypeStruct((N, MUL), jnp.float32),
        ],
    )(node_feat, node_attr, *t_args,
      W_skip, W_lin1, W_mlp3, W2_0, W2_1, W2_2)

    return (jnp.transpose(out9, (1, 2, 0)), sc)
